# baseline (speedup 1.0000x reference)
import functools

import jax
import jax.numpy as jnp
from jax import lax
from jax.experimental import pallas as pl
from jax.experimental.pallas import tpu as pltpu
from jax.experimental.pallas import tpu_sc as plsc

B = 64
N = 160000
K = 4800
KP = 4864
LANES = 16
NG = 8
GR = 8
W = 5120
NVEC = KP // LANES
MAXT0 = (N - W) // 128

_mesh = plsc.VectorSubcoreMesh(core_axis_name="c", subcore_axis_name="s")


@functools.partial(
    pl.kernel,
    out_type=jax.ShapeDtypeStruct((B, N), jnp.float32),
    mesh=_mesh,
    scratch_types=[
        pltpu.VMEM((GR, W), jnp.float32),
        pltpu.VMEM((GR, W), jnp.float32),
        pltpu.VMEM((GR, KP), jnp.int32),
        pltpu.VMEM_SHARED((GR, KP), jnp.int32),
        pltpu.VMEM_SHARED((GR, KP), jnp.int32),
        pltpu.SemaphoreType.DMA,
        pltpu.SemaphoreType.DMA,
        pltpu.SemaphoreType.DMA,
        pltpu.SemaphoreType.DMA,
        pltpu.SemaphoreType.DMA,
        pltpu.SemaphoreType.DMA,
        pltpu.SemaphoreType.DMA,
    ],
    compiler_params=pltpu.CompilerParams(
        use_tc_tiling_on_sc=True, needs_layout_passes=False),
)
def _suppress(audio_hbm, idx_hbm, out_hbm,
              slab0, slab1, idxloc, idxsp0, idxsp1,
              lsem0, lsem1, ssem0, ssem1, isem0, isem1, psem):
    sid = lax.axis_index("s")
    cid = lax.axis_index("c")
    wid = cid * 16 + sid

    slabs = (slab0, slab1)
    lsems = (lsem0, lsem1)
    ssems = (ssem0, ssem1)
    idxsps = (idxsp0, idxsp1)
    isems = (isem0, isem1)

    c0 = pl.multiple_of(jnp.minimum(wid * (W // 128), MAXT0) * 128, 128)

    zeros = jnp.zeros((LANES,), jnp.float32)

    def stage_idx(g):
        return pltpu.async_copy(
            idx_hbm.at[pl.ds(GR * g, GR), :], idxsps[g % 2], isems[g % 2])

    def slab_load(g):
        b = g % 2
        return pltpu.async_copy(
            audio_hbm.at[pl.ds(GR * g, GR), pl.ds(c0, W)], slabs[b], lsems[b])

    def slab_store(g):
        b = g % 2
        return pltpu.async_copy(
            slabs[b], out_hbm.at[pl.ds(GR * g, GR), pl.ds(c0, W)], ssems[b])

    load = slab_load(0)

    @pl.when(sid == 0)
    def _():
        stage_idx(0).wait()

    plsc.subcore_barrier()
    pltpu.sync_copy(idxsps[0], idxloc)

    @pl.when(sid == 0)
    def _():
        stage_idx(1)

    store = [None, None]

    for g in range(NG):
        b = g % 2
        load.wait()
        if g + 1 < NG:
            if store[1 - b] is not None:
                store[1 - b].wait()
                store[1 - b] = None
            load = slab_load(g + 1)
        if g > 0:
            pull.wait()

        slab = slabs[b]
        for r in range(GR):
            @plsc.parallel_loop(0, NVEC, 1, unroll=4)
            def _(k, r=r, slab=slab):
                v = idxloc[r, pl.ds(k * LANES, LANES)]
                local = v - c0
                m = local.astype(jnp.uint32) < jnp.uint32(W)
                rr = jnp.full((LANES,), r, jnp.int32)
                plsc.store_scatter(slab, [rr, local], zeros, mask=m)

        store[b] = slab_store(g)

        if g + 1 < NG:
            @pl.when(sid == 0)
            def _(g=g):
                pltpu.make_async_copy(
                    idx_hbm.at[pl.ds(GR * (g + 1), GR), :],
                    idxsps[(g + 1) % 2], isems[(g + 1) % 2]).wait()

            plsc.subcore_barrier()
            pull = pltpu.async_copy(idxsps[(g + 1) % 2], idxloc, psem)
            if g + 2 < NG:
                @pl.when(sid == 0)
                def _(g=g):
                    stage_idx(g + 2)

    for s in store:
        if s is not None:
            s.wait()


def kernel(rec_audio, attack_idx):
    idx_pad = jnp.concatenate([attack_idx, attack_idx[:, :KP - K]], axis=1)
    return _suppress(rec_audio, idx_pad)


# speedup vs baseline: 1.0156x; 1.0156x over previous
import functools

import jax
import jax.numpy as jnp
from jax import lax
from jax.experimental import pallas as pl
from jax.experimental.pallas import tpu as pltpu
from jax.experimental.pallas import tpu_sc as plsc

B = 64
N = 160000
K = 4800
KP = 4864
LANES = 16
NG = 8
GR = 8
W = 5120
NVEC = KP // LANES
MAXT0 = (N - W) // 128

_mesh = plsc.VectorSubcoreMesh(core_axis_name="c", subcore_axis_name="s")


@functools.partial(
    pl.kernel,
    out_type=jax.ShapeDtypeStruct((B, N), jnp.float32),
    mesh=_mesh,
    scratch_types=[
        pltpu.VMEM((GR, W), jnp.float32),
        pltpu.VMEM((GR, W), jnp.float32),
        pltpu.VMEM((GR, KP), jnp.int32),
        pltpu.VMEM_SHARED((GR, KP), jnp.int32),
        pltpu.VMEM_SHARED((GR, KP), jnp.int32),
        pltpu.SemaphoreType.DMA,
        pltpu.SemaphoreType.DMA,
        pltpu.SemaphoreType.DMA,
        pltpu.SemaphoreType.DMA,
        pltpu.SemaphoreType.DMA,
        pltpu.SemaphoreType.DMA,
        pltpu.SemaphoreType.DMA,
    ],
    compiler_params=pltpu.CompilerParams(
        use_tc_tiling_on_sc=True, needs_layout_passes=False),
)
def _suppress(audio_hbm, idx_hbm, out_hbm,
              slab0, slab1, idxloc, idxsp0, idxsp1,
              lsem0, lsem1, ssem0, ssem1, isem0, isem1, psem):
    sid = lax.axis_index("s")
    cid = lax.axis_index("c")
    wid = cid * 16 + sid

    slabs = (slab0, slab1)
    lsems = (lsem0, lsem1)
    ssems = (ssem0, ssem1)
    idxsps = (idxsp0, idxsp1)
    isems = (isem0, isem1)

    c0 = pl.multiple_of(jnp.minimum(wid * (W // 128), MAXT0) * 128, 128)

    zeros = jnp.zeros((LANES,), jnp.float32)

    def stage_idx(g):
        return pltpu.async_copy(
            idx_hbm.at[pl.ds(GR * g, GR), :], idxsps[g % 2], isems[g % 2])

    def slab_load(g):
        b = g % 2
        return pltpu.async_copy(
            audio_hbm.at[pl.ds(GR * g, GR), pl.ds(c0, W)], slabs[b], lsems[b])

    def slab_store(g):
        b = g % 2
        return pltpu.async_copy(
            slabs[b], out_hbm.at[pl.ds(GR * g, GR), pl.ds(c0, W)], ssems[b])

    load = slab_load(0)

    @pl.when(sid == 0)
    def _():
        stage_idx(0).wait()

    plsc.subcore_barrier()
    pltpu.sync_copy(idxsps[0], idxloc)

    @pl.when(sid == 0)
    def _():
        stage_idx(1)

    store = [None, None]

    for g in range(NG):
        b = g % 2
        load.wait()
        if g + 1 < NG:
            if store[1 - b] is not None:
                store[1 - b].wait()
                store[1 - b] = None
            load = slab_load(g + 1)
        if g > 0:
            pull.wait()

        slab = slabs[b]
        for r in range(GR):
            @plsc.parallel_loop(0, NVEC, 1, unroll=8)
            def _(k, r=r, slab=slab):
                v = idxloc[r, pl.ds(k * LANES, LANES)]
                local = v - c0
                m = local.astype(jnp.uint32) < jnp.uint32(W)
                rr = jnp.full((LANES,), r, jnp.int32)
                plsc.store_scatter(slab, [rr, local], zeros, mask=m)

        store[b] = slab_store(g)

        if g + 1 < NG:
            @pl.when(sid == 0)
            def _(g=g):
                pltpu.make_async_copy(
                    idx_hbm.at[pl.ds(GR * (g + 1), GR), :],
                    idxsps[(g + 1) % 2], isems[(g + 1) % 2]).wait()

            plsc.subcore_barrier()
            pull = pltpu.async_copy(idxsps[(g + 1) % 2], idxloc, psem)
            if g + 2 < NG:
                @pl.when(sid == 0)
                def _(g=g):
                    stage_idx(g + 2)

    for s in store:
        if s is not None:
            s.wait()


def kernel(rec_audio, attack_idx):
    idx_pad = jnp.concatenate([attack_idx, attack_idx[:, :KP - K]], axis=1)
    return _suppress(rec_audio, idx_pad)
